# baseline (speedup 1.0000x reference)
import functools

import jax
import jax.numpy as jnp
from jax import lax
from jax.experimental import pallas as pl
from jax.experimental.pallas import tpu as pltpu
from jax.experimental.pallas import tpu_sc as plsc

_FT_MAIN = 2048


def _build_ft_gather(rows_per_worker, workers_per_core, n_rows):
    mesh = plsc.VectorSubcoreMesh(core_axis_name="c", subcore_axis_name="s")

    @functools.partial(
        pl.kernel,
        mesh=mesh,
        out_type=jax.ShapeDtypeStruct((n_rows, _FT_MAIN), jnp.float32),
        scratch_types=[
            pltpu.VMEM((rows_per_worker,), jnp.int32),
            pltpu.VMEM((rows_per_worker, _FT_MAIN), jnp.float32),
            pltpu.SemaphoreType.DMA,
        ],
    )
    def ft_gather(ft_w_hbm, gidx_hbm, out_hbm, idx_v, rows_v, sem):
        c = lax.axis_index("c")
        s = lax.axis_index("s")
        w = jnp.minimum(c * workers_per_core + s, 2 * workers_per_core - 1)

        @pl.when(s < workers_per_core)
        def _gather():
            base = w * rows_per_worker
            pltpu.sync_copy(gidx_hbm.at[pl.ds(base, rows_per_worker)], idx_v)
            pltpu.async_copy(
                ft_w_hbm.at[idx_v, pl.ds(0, _FT_MAIN)], rows_v, sem).wait()
            pltpu.sync_copy(rows_v, out_hbm.at[pl.ds(base, rows_per_worker)])

    return ft_gather


def _tail_body(rows_ref, ftb_ref, stm_ref, cnt_ref, psq_ref, hot_ref, r0_ref,
               l1w_ref, l1b_ref, l2w_ref, l2b_ref, ow_ref, ob_ref, out_ref):
    d = _FT_MAIN
    f = rows_ref.shape[0] // 2
    stm = stm_ref[0, 0]
    sum_w = (jnp.sum(rows_ref[:f, :], axis=0, keepdims=True)
             - cnt_ref[0, 0] * r0_ref[...])
    sum_b = (jnp.sum(rows_ref[f:, :], axis=0, keepdims=True)
             - cnt_ref[0, 1] * r0_ref[...])
    w = sum_w + ftb_ref[:, :d]
    b = sum_b + ftb_ref[:, :d]
    acc = stm * jnp.concatenate([w, b], axis=1) \
        + (1.0 - stm) * jnp.concatenate([b, w], axis=1)
    l1 = jnp.clip(acc, 0.0, 1.0)
    h1 = lax.dot_general(l1, l1w_ref[...], (((1,), (1,)), ((), ())),
                         preferred_element_type=jnp.float32) + l1b_ref[...]
    h2 = lax.dot_general(h1, l2w_ref[...], (((1,), (1,)), ((), ())),
                         preferred_element_type=jnp.float32) + l2b_ref[...]
    h2 = jnp.clip(h2, 0.0, 1.0)
    out = jnp.sum(h2 * ow_ref[...], axis=1, keepdims=True) + ob_ref[...]
    psqt_diff = jnp.sum(psq_ref[...] * hot_ref[...])
    out_ref[...] = out + psqt_diff * (stm - 0.5)


def kernel(white_features, black_features, stm, piece_counts,
           ft_W, ft_b, l1_W, l1_b, l2_W, l2_b, out_W, out_b):
    del piece_counts
    in_dim, d_cols = ft_W.shape
    bsz, f = white_features.shape

    widx = white_features.reshape(-1).astype(jnp.int32)
    bidx = black_features.reshape(-1).astype(jnp.int32)
    idx_all = jnp.concatenate([widx, bidx])
    safe_idx = jnp.maximum(idx_all, 0)
    counts = jnp.stack([jnp.sum(widx < 0), jnp.sum(bidx < 0)]
                       ).astype(jnp.float32).reshape(1, 2)

    wsign = jnp.where(widx >= 0, 1.0, 0.0)
    bsign = jnp.where(bidx >= 0, -1.0, 0.0)
    hot = jnp.zeros((in_dim,), jnp.float32)
    hot = hot.at[jnp.maximum(widx, 0)].add(wsign)
    hot = hot.at[jnp.maximum(bidx, 0)].add(bsign)
    hot = hot.reshape(in_dim // 128, 128)
    psq_col = ft_W[:, d_cols - 1].reshape(in_dim // 128, 128)
    row0_main = ft_W[0:1, :_FT_MAIN]

    rows_per_worker = 8
    workers_per_core = f // rows_per_worker

    ft_gather = _build_ft_gather(rows_per_worker, workers_per_core, 2 * f)
    rows = ft_gather(ft_W, safe_idx)

    vmem = pl.BlockSpec(memory_space=pltpu.VMEM)
    smem = pl.BlockSpec(memory_space=pltpu.SMEM)
    out = pl.pallas_call(
        _tail_body,
        out_shape=jax.ShapeDtypeStruct((1, 1), jnp.float32),
        in_specs=[vmem, vmem, smem, smem, vmem, vmem, vmem,
                  vmem, vmem, vmem, vmem, vmem, vmem],
        out_specs=vmem,
    )(
        rows, ft_b.reshape(1, -1), stm.reshape(1, 1), counts,
        psq_col, hot, row0_main,
        l1_W, l1_b.reshape(1, -1), l2_W, l2_b.reshape(1, -1),
        out_W, out_b.reshape(1, 1),
    )
    return out.reshape(bsz)


# speedup vs baseline: 1.0820x; 1.0820x over previous
import functools

import jax
import jax.numpy as jnp
from jax import lax
from jax.experimental import pallas as pl
from jax.experimental.pallas import tpu as pltpu
from jax.experimental.pallas import tpu_sc as plsc

_FT_MAIN = 2048


def _build_ft_gather(rows_per_worker, workers_per_core, n_rows):
    mesh = plsc.VectorSubcoreMesh(core_axis_name="c", subcore_axis_name="s")

    @functools.partial(
        pl.kernel,
        mesh=mesh,
        out_type=(
            jax.ShapeDtypeStruct((n_rows, _FT_MAIN), jnp.float32),
            jax.ShapeDtypeStruct((n_rows, 128), jnp.float32),
        ),
        scratch_types=[
            pltpu.VMEM((rows_per_worker,), jnp.int32),
            pltpu.VMEM((rows_per_worker,), jnp.int32),
            pltpu.VMEM((rows_per_worker, _FT_MAIN), jnp.float32),
            pltpu.VMEM((rows_per_worker, 128), jnp.float32),
            pltpu.SemaphoreType.DMA,
        ],
    )
    def ft_gather(ft_w_hbm, psq_hbm, gidx_hbm, pidx_hbm, out_hbm, outp_hbm,
                  idx_v, pidx_v, rows_v, prow_v, sem):
        c = lax.axis_index("c")
        s = lax.axis_index("s")
        w = jnp.minimum(c * workers_per_core + s, 2 * workers_per_core - 1)

        @pl.when(s < workers_per_core)
        def _gather():
            base = w * rows_per_worker
            pltpu.sync_copy(gidx_hbm.at[pl.ds(base, rows_per_worker)], idx_v)
            pltpu.sync_copy(pidx_hbm.at[pl.ds(base, rows_per_worker)], pidx_v)
            cp_main = pltpu.async_copy(
                ft_w_hbm.at[idx_v, pl.ds(0, _FT_MAIN)], rows_v, sem)
            cp_psq = pltpu.async_copy(psq_hbm.at[pidx_v], prow_v, sem)
            cp_main.wait()
            cp_psq.wait()
            pltpu.sync_copy(rows_v, out_hbm.at[pl.ds(base, rows_per_worker)])
            pltpu.sync_copy(prow_v, outp_hbm.at[pl.ds(base, rows_per_worker)])

    return ft_gather


def _tail_body(rows_ref, prow_ref, sel_ref, ftb_ref, stm_ref, cnt_ref, r0_ref,
               l1w_ref, l1b_ref, l2w_ref, l2b_ref, ow_ref, ob_ref, out_ref):
    d = _FT_MAIN
    f = rows_ref.shape[0] // 2
    stm = stm_ref[0, 0]
    sum_w = (jnp.sum(rows_ref[:f, :], axis=0, keepdims=True)
             - cnt_ref[0, 0] * r0_ref[...])
    sum_b = (jnp.sum(rows_ref[f:, :], axis=0, keepdims=True)
             - cnt_ref[0, 1] * r0_ref[...])
    w = sum_w + ftb_ref[:, :d]
    b = sum_b + ftb_ref[:, :d]
    acc = stm * jnp.concatenate([w, b], axis=1) \
        + (1.0 - stm) * jnp.concatenate([b, w], axis=1)
    l1 = jnp.clip(acc, 0.0, 1.0)
    h1 = lax.dot_general(l1, l1w_ref[...], (((1,), (1,)), ((), ())),
                         preferred_element_type=jnp.float32) + l1b_ref[...]
    h2 = lax.dot_general(h1, l2w_ref[...], (((1,), (1,)), ((), ())),
                         preferred_element_type=jnp.float32) + l2b_ref[...]
    h2 = jnp.clip(h2, 0.0, 1.0)
    out = jnp.sum(h2 * ow_ref[...], axis=1, keepdims=True) + ob_ref[...]
    psqt_diff = jnp.sum(prow_ref[...] * sel_ref[...])
    out_ref[...] = out + psqt_diff * (stm - 0.5)


def kernel(white_features, black_features, stm, piece_counts,
           ft_W, ft_b, l1_W, l1_b, l2_W, l2_b, out_W, out_b):
    del piece_counts
    in_dim, d_cols = ft_W.shape
    bsz, f = white_features.shape

    widx = white_features.reshape(-1).astype(jnp.int32)
    bidx = black_features.reshape(-1).astype(jnp.int32)
    idx_all = jnp.concatenate([widx, bidx])
    safe_idx = jnp.maximum(idx_all, 0)
    counts = jnp.stack([jnp.sum(widx < 0), jnp.sum(bidx < 0)]
                       ).astype(jnp.float32).reshape(1, 2)

    psq_tab = ft_W[:, d_cols - 1].reshape(in_dim // 128, 128)
    psq_row_idx = safe_idx // 128
    sign = jnp.where(idx_all >= 0,
                     jnp.where(jnp.arange(2 * f) < f, 1.0, -1.0), 0.0)
    col_sel = (jnp.arange(128, dtype=jnp.int32)[None, :]
               == (safe_idx % 128)[:, None]).astype(jnp.float32)
    col_sel = col_sel * sign[:, None]

    rows_per_worker = 8
    workers_per_core = f // rows_per_worker

    ft_gather = _build_ft_gather(rows_per_worker, workers_per_core, 2 * f)
    rows, prows = ft_gather(ft_W, psq_tab, safe_idx, psq_row_idx)

    vmem = pl.BlockSpec(memory_space=pltpu.VMEM)
    smem = pl.BlockSpec(memory_space=pltpu.SMEM)
    out = pl.pallas_call(
        _tail_body,
        out_shape=jax.ShapeDtypeStruct((1, 1), jnp.float32),
        in_specs=[vmem, vmem, vmem, vmem, smem, smem, vmem,
                  vmem, vmem, vmem, vmem, vmem, vmem],
        out_specs=vmem,
    )(
        rows, prows, col_sel, ft_b.reshape(1, -1), stm.reshape(1, 1), counts,
        ft_W[0:1, :_FT_MAIN],
        l1_W, l1_b.reshape(1, -1), l2_W, l2_b.reshape(1, -1),
        out_W, out_b.reshape(1, 1),
    )
    return out.reshape(bsz)


# speedup vs baseline: 3.2965x; 3.0467x over previous
import functools

import jax
import jax.numpy as jnp
from jax import lax
from jax.experimental import pallas as pl
from jax.experimental.pallas import tpu as pltpu
from jax.experimental.pallas import tpu_sc as plsc

_FT_MAIN = 2048
_BAND = 64
_LANES = 16


def _build_ft_gather(n_feat):
    mesh = plsc.VectorSubcoreMesh(core_axis_name="c", subcore_axis_name="s")
    psq_per_worker = 8
    psq_workers = n_feat // psq_per_worker

    @functools.partial(
        pl.kernel,
        mesh=mesh,
        compiler_params=pltpu.CompilerParams(needs_layout_passes=False),
        out_type=(
            jax.ShapeDtypeStruct((_FT_MAIN,), jnp.float32),
            jax.ShapeDtypeStruct((_FT_MAIN,), jnp.float32),
            jax.ShapeDtypeStruct((n_feat, 128), jnp.float32),
        ),
        scratch_types=[
            pltpu.VMEM((n_feat,), jnp.int32),
            pltpu.VMEM((n_feat,), jnp.int32),
            pltpu.VMEM((n_feat,), jnp.float32),
            pltpu.VMEM((n_feat,), jnp.float32),
            pltpu.VMEM((2, _BAND, 128), jnp.float32),
            pltpu.VMEM((_BAND,), jnp.float32),
            pltpu.VMEM((_BAND,), jnp.float32),
            pltpu.VMEM((psq_per_worker,), jnp.int32),
            pltpu.VMEM((psq_per_worker, 128), jnp.float32),
            pltpu.SemaphoreType.DMA,
            pltpu.SemaphoreType.DMA,
            pltpu.SemaphoreType.DMA,
        ],
    )
    def ft_gather(t_hbm, psq_hbm, base_hbm, lane_hbm, wgt_w_hbm, wgt_b_hbm,
                  pidx_hbm, out_w, out_b, out_psq,
                  base_v, lane_v, wgt_w_v, wgt_b_v, buf_v, acc_w_v, acc_b_v,
                  pidx_v, prow_v, sem0, sem1, psem):
        c = lax.axis_index("c")
        s = lax.axis_index("s")
        wid = c * 16 + s
        c0 = pl.multiple_of(wid * _BAND, _BAND)

        pltpu.sync_copy(base_hbm, base_v)
        pltpu.sync_copy(lane_hbm, lane_v)
        pltpu.sync_copy(wgt_w_hbm, wgt_w_v)
        pltpu.sync_copy(wgt_b_hbm, wgt_b_v)

        @pl.when(wid < psq_workers)
        def _psq_start():
            pbase = wid * psq_per_worker
            pltpu.sync_copy(pidx_hbm.at[pl.ds(pbase, psq_per_worker)], pidx_v)
            pltpu.async_copy(psq_hbm.at[pidx_v], prow_v, psem)

        sems = (sem0, sem1)

        def _window_copy(base_j, buf_slot):
            base_j = pl.multiple_of(base_j, 128)
            return pltpu.async_copy(
                t_hbm.at[pl.ds(c0, _BAND), pl.ds(base_j, 128)],
                buf_v.at[buf_slot], sems[buf_slot])

        n_ch = n_feat // _LANES
        base_ch = [base_v[pl.ds(k * _LANES, _LANES)] for k in range(n_ch)]
        lane_ch = [lane_v[pl.ds(k * _LANES, _LANES)] for k in range(n_ch)]
        wgt_w_ch = [wgt_w_v[pl.ds(k * _LANES, _LANES)] for k in range(n_ch)]
        wgt_b_ch = [wgt_b_v[pl.ds(k * _LANES, _LANES)] for k in range(n_ch)]

        zero = jnp.zeros((_LANES,), jnp.float32)
        acc_w = [zero] * (_BAND // _LANES)
        acc_b = [zero] * (_BAND // _LANES)
        cp = _window_copy(base_ch[0][0], 0)
        for j in range(n_feat):
            if j + 1 < n_feat:
                jn = j + 1
                cp_next = _window_copy(base_ch[jn // _LANES][jn % _LANES],
                                       jn % 2)
            cp.wait()
            lane_j = lane_ch[j // _LANES][j % _LANES]
            cols = jnp.full((_LANES,), lane_j, jnp.int32)
            wch = wgt_w_ch if j < n_feat // 2 else wgt_b_ch
            wgt_j = wch[j // _LANES][j % _LANES]
            acc = acc_w if j < n_feat // 2 else acc_b
            for k in range(_BAND // _LANES):
                rows = lax.iota(jnp.int32, _LANES) + (k * _LANES)
                vals = plsc.load_gather(buf_v.at[j % 2], [rows, cols])
                acc[k] = acc[k] + vals * wgt_j
            if j + 1 < n_feat:
                cp = cp_next
        for k in range(_BAND // _LANES):
            acc_w_v[pl.ds(k * _LANES, _LANES)] = acc_w[k]
            acc_b_v[pl.ds(k * _LANES, _LANES)] = acc_b[k]
        pltpu.sync_copy(acc_w_v, out_w.at[pl.ds(c0, _BAND)])
        pltpu.sync_copy(acc_b_v, out_b.at[pl.ds(c0, _BAND)])

        @pl.when(wid < psq_workers)
        def _psq_finish():
            pbase = wid * psq_per_worker
            pltpu.make_async_copy(
                psq_hbm.at[pidx_v], prow_v, psem).wait()
            pltpu.sync_copy(
                prow_v, out_psq.at[pl.ds(pbase, psq_per_worker)])

    return ft_gather


def _tail_body(sw_ref, sb_ref, prow_ref, sel_ref, ftb_ref, stm_ref,
               l1w_ref, l1b_ref, l2w_ref, l2b_ref, ow_ref, ob_ref, out_ref):
    d = _FT_MAIN
    stm = stm_ref[0, 0]
    w = sw_ref[...] + ftb_ref[:, :d]
    b = sb_ref[...] + ftb_ref[:, :d]
    acc = stm * jnp.concatenate([w, b], axis=1) \
        + (1.0 - stm) * jnp.concatenate([b, w], axis=1)
    l1 = jnp.clip(acc, 0.0, 1.0)
    h1 = lax.dot_general(l1, l1w_ref[...], (((1,), (1,)), ((), ())),
                         preferred_element_type=jnp.float32) + l1b_ref[...]
    h2 = lax.dot_general(h1, l2w_ref[...], (((1,), (1,)), ((), ())),
                         preferred_element_type=jnp.float32) + l2b_ref[...]
    h2 = jnp.clip(h2, 0.0, 1.0)
    out = jnp.sum(h2 * ow_ref[...], axis=1, keepdims=True) + ob_ref[...]
    psqt_diff = jnp.sum(prow_ref[...] * sel_ref[...])
    out_ref[...] = out + psqt_diff * (stm - 0.5)


def kernel(white_features, black_features, stm, piece_counts,
           ft_W, ft_b, l1_W, l1_b, l2_W, l2_b, out_W, out_b):
    del piece_counts
    in_dim, d_cols = ft_W.shape
    bsz, f = white_features.shape
    n_feat = 2 * f

    t_tab = ft_W.T

    widx = white_features.reshape(-1).astype(jnp.int32)
    bidx = black_features.reshape(-1).astype(jnp.int32)
    idx_all = jnp.concatenate([widx, bidx])
    safe_idx = jnp.maximum(idx_all, 0)
    mask = (idx_all >= 0).astype(jnp.float32)
    base = (safe_idx // 128) * 128
    lane = safe_idx % 128
    wgt_w = jnp.where(jnp.arange(n_feat) < f, mask, 0.0)
    wgt_b = jnp.where(jnp.arange(n_feat) >= f, mask, 0.0)

    psq_tab = t_tab[d_cols - 1].reshape(in_dim // 128, 128)
    psq_row_idx = safe_idx // 128
    sign = jnp.where(jnp.arange(n_feat) < f, 1.0, -1.0) * mask
    col_sel = (jnp.arange(128, dtype=jnp.int32)[None, :]
               == lane[:, None]).astype(jnp.float32) * sign[:, None]

    ft_gather = _build_ft_gather(n_feat)
    sum_w, sum_b, prows = ft_gather(
        t_tab, psq_tab, base, lane, wgt_w, wgt_b, psq_row_idx)

    vmem = pl.BlockSpec(memory_space=pltpu.VMEM)
    smem = pl.BlockSpec(memory_space=pltpu.SMEM)
    out = pl.pallas_call(
        _tail_body,
        out_shape=jax.ShapeDtypeStruct((1, 1), jnp.float32),
        in_specs=[vmem, vmem, vmem, vmem, vmem, smem,
                  vmem, vmem, vmem, vmem, vmem, vmem],
        out_specs=vmem,
    )(
        sum_w.reshape(1, -1), sum_b.reshape(1, -1), prows, col_sel,
        ft_b.reshape(1, -1), stm.reshape(1, 1),
        l1_W, l1_b.reshape(1, -1), l2_W, l2_b.reshape(1, -1),
        out_W, out_b.reshape(1, 1),
    )
    return out.reshape(bsz)


# speedup vs baseline: 3.8265x; 1.1608x over previous
import functools

import jax
import jax.numpy as jnp
from jax import lax
from jax.experimental import pallas as pl
from jax.experimental.pallas import tpu as pltpu
from jax.experimental.pallas import tpu_sc as plsc

_FT_MAIN = 2048
_BAND = 64
_LANES = 16
_NBUF = 4


def _build_ft_gather(n_feat):
    mesh = plsc.VectorSubcoreMesh(core_axis_name="c", subcore_axis_name="s")
    psq_per_worker = 8
    psq_workers = n_feat // psq_per_worker

    @functools.partial(
        pl.kernel,
        mesh=mesh,
        compiler_params=pltpu.CompilerParams(needs_layout_passes=False),
        out_type=(
            jax.ShapeDtypeStruct((_FT_MAIN,), jnp.float32),
            jax.ShapeDtypeStruct((_FT_MAIN,), jnp.float32),
            jax.ShapeDtypeStruct((n_feat, 128), jnp.float32),
        ),
        scratch_types=[
            pltpu.VMEM((n_feat,), jnp.int32),
            pltpu.VMEM((n_feat,), jnp.int32),
            pltpu.VMEM((n_feat,), jnp.float32),
            pltpu.VMEM((n_feat,), jnp.float32),
            pltpu.VMEM((_NBUF, _BAND, 128), jnp.float32),
            pltpu.VMEM((_BAND,), jnp.float32),
            pltpu.VMEM((_BAND,), jnp.float32),
            pltpu.VMEM((psq_per_worker,), jnp.int32),
            pltpu.VMEM((psq_per_worker, 128), jnp.float32),
            [pltpu.SemaphoreType.DMA] * _NBUF,
            pltpu.SemaphoreType.DMA,
        ],
    )
    def ft_gather(t_hbm, psq_hbm, base_hbm, lane_hbm, wgt_w_hbm, wgt_b_hbm,
                  pidx_hbm, out_w, out_b, out_psq,
                  base_v, lane_v, wgt_w_v, wgt_b_v, buf_v, acc_w_v, acc_b_v,
                  pidx_v, prow_v, sems, psem):
        c = lax.axis_index("c")
        s = lax.axis_index("s")
        wid = c * 16 + s
        c0 = pl.multiple_of(wid * _BAND, _BAND)

        pltpu.sync_copy(base_hbm, base_v)
        pltpu.sync_copy(lane_hbm, lane_v)
        pltpu.sync_copy(wgt_w_hbm, wgt_w_v)
        pltpu.sync_copy(wgt_b_hbm, wgt_b_v)

        @pl.when(wid < psq_workers)
        def _psq_start():
            pbase = wid * psq_per_worker
            pltpu.sync_copy(pidx_hbm.at[pl.ds(pbase, psq_per_worker)], pidx_v)
            pltpu.async_copy(psq_hbm.at[pidx_v], prow_v, psem)

        def _fetch_scalar(ref, j):
            return plsc.load_gather(ref, [jnp.full((_LANES,), j, jnp.int32)])[0]

        def _window_copy(j, slot):
            base_j = pl.multiple_of(_fetch_scalar(base_v, j), 128)
            pltpu.async_copy(
                t_hbm.at[pl.ds(c0, _BAND), pl.ds(base_j, 128)],
                buf_v.at[slot], sems[slot])

        zero = jnp.zeros((_LANES,), jnp.float32)
        for k in range(_BAND // _LANES):
            acc_w_v[pl.ds(k * _LANES, _LANES)] = zero
            acc_b_v[pl.ds(k * _LANES, _LANES)] = zero

        for t in range(_NBUF):
            _window_copy(t, t)

        def _outer(i, _):
            j0 = i * _NBUF
            for t in range(_NBUF):
                j = j0 + t
                pltpu.make_async_copy(
                    t_hbm.at[pl.ds(c0, _BAND), pl.ds(0, 128)],
                    buf_v.at[t], sems[t]).wait()
                lane_j = _fetch_scalar(lane_v, j)
                cols = jnp.full((_LANES,), lane_j, jnp.int32)
                wgt_w_j = _fetch_scalar(wgt_w_v, j)
                wgt_b_j = _fetch_scalar(wgt_b_v, j)
                for k in range(_BAND // _LANES):
                    rows = lax.iota(jnp.int32, _LANES) + (k * _LANES)
                    vals = plsc.load_gather(buf_v.at[t], [rows, cols])
                    sl = pl.ds(k * _LANES, _LANES)
                    acc_w_v[sl] = acc_w_v[sl] + vals * wgt_w_j
                    acc_b_v[sl] = acc_b_v[sl] + vals * wgt_b_j

                @pl.when(j + _NBUF < n_feat)
                def _refill():
                    _window_copy(j + _NBUF, t)
            return _

        lax.fori_loop(0, n_feat // _NBUF, _outer, None)
        pltpu.sync_copy(acc_w_v, out_w.at[pl.ds(c0, _BAND)])
        pltpu.sync_copy(acc_b_v, out_b.at[pl.ds(c0, _BAND)])

        @pl.when(wid < psq_workers)
        def _psq_finish():
            pbase = wid * psq_per_worker
            pltpu.make_async_copy(
                psq_hbm.at[pidx_v], prow_v, psem).wait()
            pltpu.sync_copy(
                prow_v, out_psq.at[pl.ds(pbase, psq_per_worker)])

    return ft_gather


def _tail_body(sw_ref, sb_ref, prow_ref, sel_ref, ftb_ref, stm_ref,
               l1w_ref, l1b_ref, l2w_ref, l2b_ref, ow_ref, ob_ref, out_ref):
    d = _FT_MAIN
    stm = stm_ref[0, 0]
    w = sw_ref[...] + ftb_ref[:, :d]
    b = sb_ref[...] + ftb_ref[:, :d]
    acc = stm * jnp.concatenate([w, b], axis=1) \
        + (1.0 - stm) * jnp.concatenate([b, w], axis=1)
    l1 = jnp.clip(acc, 0.0, 1.0)
    h1 = lax.dot_general(l1, l1w_ref[...], (((1,), (1,)), ((), ())),
                         preferred_element_type=jnp.float32) + l1b_ref[...]
    h2 = lax.dot_general(h1, l2w_ref[...], (((1,), (1,)), ((), ())),
                         preferred_element_type=jnp.float32) + l2b_ref[...]
    h2 = jnp.clip(h2, 0.0, 1.0)
    out = jnp.sum(h2 * ow_ref[...], axis=1, keepdims=True) + ob_ref[...]
    psqt_diff = jnp.sum(prow_ref[...] * sel_ref[...])
    out_ref[...] = out + psqt_diff * (stm - 0.5)


def kernel(white_features, black_features, stm, piece_counts,
           ft_W, ft_b, l1_W, l1_b, l2_W, l2_b, out_W, out_b):
    del piece_counts
    in_dim, d_cols = ft_W.shape
    bsz, f = white_features.shape
    n_feat = 2 * f

    t_tab = ft_W.T

    widx = white_features.reshape(-1).astype(jnp.int32)
    bidx = black_features.reshape(-1).astype(jnp.int32)
    idx_all = jnp.concatenate([widx, bidx])
    safe_idx = jnp.maximum(idx_all, 0)
    mask = (idx_all >= 0).astype(jnp.float32)
    base = (safe_idx // 128) * 128
    lane = safe_idx % 128
    wgt_w = jnp.where(jnp.arange(n_feat) < f, mask, 0.0)
    wgt_b = jnp.where(jnp.arange(n_feat) >= f, mask, 0.0)

    psq_tab = t_tab[d_cols - 1].reshape(in_dim // 128, 128)
    psq_row_idx = safe_idx // 128
    sign = jnp.where(jnp.arange(n_feat) < f, 1.0, -1.0) * mask
    col_sel = (jnp.arange(128, dtype=jnp.int32)[None, :]
               == lane[:, None]).astype(jnp.float32) * sign[:, None]

    ft_gather = _build_ft_gather(n_feat)
    sum_w, sum_b, prows = ft_gather(
        t_tab, psq_tab, base, lane, wgt_w, wgt_b, psq_row_idx)

    vmem = pl.BlockSpec(memory_space=pltpu.VMEM)
    smem = pl.BlockSpec(memory_space=pltpu.SMEM)
    out = pl.pallas_call(
        _tail_body,
        out_shape=jax.ShapeDtypeStruct((1, 1), jnp.float32),
        in_specs=[vmem, vmem, vmem, vmem, vmem, smem,
                  vmem, vmem, vmem, vmem, vmem, vmem],
        out_specs=vmem,
    )(
        sum_w.reshape(1, -1), sum_b.reshape(1, -1), prows, col_sel,
        ft_b.reshape(1, -1), stm.reshape(1, 1),
        l1_W, l1_b.reshape(1, -1), l2_W, l2_b.reshape(1, -1),
        out_W, out_b.reshape(1, 1),
    )
    return out.reshape(bsz)


# speedup vs baseline: 4.1401x; 1.0820x over previous
import functools

import jax
import jax.numpy as jnp
from jax import lax
from jax.experimental import pallas as pl
from jax.experimental.pallas import tpu as pltpu
from jax.experimental.pallas import tpu_sc as plsc

_FT_MAIN = 2048
_BAND = 64
_LANES = 16
_NBUF = 4


def _build_ft_gather(n_feat):
    mesh = plsc.VectorSubcoreMesh(core_axis_name="c", subcore_axis_name="s")
    psq_per_worker = 8
    psq_workers = n_feat // psq_per_worker

    @functools.partial(
        pl.kernel,
        mesh=mesh,
        compiler_params=pltpu.CompilerParams(needs_layout_passes=False),
        out_type=(
            jax.ShapeDtypeStruct((_FT_MAIN,), jnp.float32),
            jax.ShapeDtypeStruct((_FT_MAIN,), jnp.float32),
            jax.ShapeDtypeStruct((n_feat, 128), jnp.float32),
        ),
        scratch_types=[
            pltpu.VMEM((n_feat,), jnp.int32),
            pltpu.VMEM((n_feat,), jnp.int32),
            pltpu.VMEM((n_feat,), jnp.int32),
            pltpu.VMEM((n_feat,), jnp.float32),
            pltpu.VMEM((n_feat,), jnp.float32),
            pltpu.VMEM((_NBUF, _BAND, 128), jnp.float32),
            pltpu.VMEM((_BAND,), jnp.float32),
            pltpu.VMEM((_BAND,), jnp.float32),
            pltpu.VMEM((psq_per_worker, 128), jnp.float32),
            [pltpu.SemaphoreType.DMA] * _NBUF,
            pltpu.SemaphoreType.DMA,
        ],
    )
    def ft_gather(t_hbm, idx_hbm, out_w, out_b, out_psq,
                  idx_v, base_v, lane_v, wgt_w_v, wgt_b_v, buf_v,
                  acc_w_v, acc_b_v, prow_v, sems, psem):
        c = lax.axis_index("c")
        s = lax.axis_index("s")
        wid = c * 16 + s
        c0 = pl.multiple_of(wid * _BAND, _BAND)
        psq_last_row = t_hbm.shape[0] - 1

        pltpu.sync_copy(idx_hbm, idx_v)
        n_ch = n_feat // _LANES
        for k in range(n_ch):
            sl = pl.ds(k * _LANES, _LANES)
            ch = idx_v[sl]
            safe = jnp.maximum(ch, 0)
            maskf = jnp.where(ch >= 0, 1.0, 0.0).astype(jnp.float32)
            zerof = jnp.zeros((_LANES,), jnp.float32)
            base_v[sl] = safe & ~jnp.int32(127)
            lane_v[sl] = safe & jnp.int32(127)
            wgt_w_v[sl] = maskf if k < n_ch // 2 else zerof
            wgt_b_v[sl] = zerof if k < n_ch // 2 else maskf

        def _fetch_scalar(ref, j):
            return plsc.load_gather(ref, [jnp.full((_LANES,), j, jnp.int32)])[0]

        def _window_copy(j, slot):
            base_j = pl.multiple_of(_fetch_scalar(base_v, j), 128)
            pltpu.async_copy(
                t_hbm.at[pl.ds(c0, _BAND), pl.ds(base_j, 128)],
                buf_v.at[slot], sems[slot])

        zero = jnp.zeros((_LANES,), jnp.float32)
        for k in range(_BAND // _LANES):
            acc_w_v[pl.ds(k * _LANES, _LANES)] = zero
            acc_b_v[pl.ds(k * _LANES, _LANES)] = zero

        is_psq = s < psq_workers // 2
        pbase = (c * (psq_workers // 2) + s) * psq_per_worker

        @pl.when(is_psq)
        def _psq_start():
            for i in range(psq_per_worker):
                pb = pl.multiple_of(_fetch_scalar(base_v, pbase + i), 128)
                pltpu.async_copy(
                    t_hbm.at[pl.ds(psq_last_row, 1), pl.ds(pb, 128)],
                    prow_v.at[pl.ds(i, 1)], psem)

        for t in range(_NBUF):
            _window_copy(t, t)

        def _outer(i, _):
            j0 = i * _NBUF
            for t in range(_NBUF):
                j = j0 + t
                pltpu.make_async_copy(
                    t_hbm.at[pl.ds(c0, _BAND), pl.ds(0, 128)],
                    buf_v.at[t], sems[t]).wait()
                lane_j = _fetch_scalar(lane_v, j)
                cols = jnp.full((_LANES,), lane_j, jnp.int32)
                wgt_w_j = _fetch_scalar(wgt_w_v, j)
                wgt_b_j = _fetch_scalar(wgt_b_v, j)
                for k in range(_BAND // _LANES):
                    rows = lax.iota(jnp.int32, _LANES) + (k * _LANES)
                    vals = plsc.load_gather(buf_v.at[t], [rows, cols])
                    sl = pl.ds(k * _LANES, _LANES)
                    acc_w_v[sl] = acc_w_v[sl] + vals * wgt_w_j
                    acc_b_v[sl] = acc_b_v[sl] + vals * wgt_b_j

                @pl.when(j + _NBUF < n_feat)
                def _refill():
                    _window_copy(j + _NBUF, t)
            return _

        lax.fori_loop(0, n_feat // _NBUF, _outer, None)
        pltpu.sync_copy(acc_w_v, out_w.at[pl.ds(c0, _BAND)])
        pltpu.sync_copy(acc_b_v, out_b.at[pl.ds(c0, _BAND)])

        @pl.when(is_psq)
        def _psq_finish():
            for i in range(psq_per_worker):
                pltpu.make_async_copy(
                    t_hbm.at[pl.ds(psq_last_row, 1), pl.ds(0, 128)],
                    prow_v.at[pl.ds(i, 1)], psem).wait()
            pltpu.sync_copy(
                prow_v, out_psq.at[pl.ds(pbase, psq_per_worker)])

    return ft_gather


def _tail_body(sw_ref, sb_ref, prow_ref, sel_ref, ftb_ref, stm_ref,
               l1w_ref, l1b_ref, l2w_ref, l2b_ref, ow_ref, ob_ref, out_ref):
    d = _FT_MAIN
    stm = stm_ref[0, 0]
    w = sw_ref[...] + ftb_ref[:, :d]
    b = sb_ref[...] + ftb_ref[:, :d]
    acc = stm * jnp.concatenate([w, b], axis=1) \
        + (1.0 - stm) * jnp.concatenate([b, w], axis=1)
    l1 = jnp.clip(acc, 0.0, 1.0)
    h1 = lax.dot_general(l1, l1w_ref[...], (((1,), (1,)), ((), ())),
                         preferred_element_type=jnp.float32) + l1b_ref[...]
    h2 = lax.dot_general(h1, l2w_ref[...], (((1,), (1,)), ((), ())),
                         preferred_element_type=jnp.float32) + l2b_ref[...]
    h2 = jnp.clip(h2, 0.0, 1.0)
    out = jnp.sum(h2 * ow_ref[...], axis=1, keepdims=True) + ob_ref[...]
    psqt_diff = jnp.sum(prow_ref[...] * sel_ref[...])
    out_ref[...] = out + psqt_diff * (stm - 0.5)


def kernel(white_features, black_features, stm, piece_counts,
           ft_W, ft_b, l1_W, l1_b, l2_W, l2_b, out_W, out_b):
    del piece_counts
    in_dim, d_cols = ft_W.shape
    bsz, f = white_features.shape
    n_feat = 2 * f

    t_tab = ft_W.T

    widx = white_features.reshape(-1).astype(jnp.int32)
    bidx = black_features.reshape(-1).astype(jnp.int32)
    idx_all = jnp.concatenate([widx, bidx])
    safe_idx = jnp.maximum(idx_all, 0)
    mask = (idx_all >= 0).astype(jnp.float32)
    lane = safe_idx % 128

    sign = jnp.where(jnp.arange(n_feat) < f, 1.0, -1.0) * mask
    col_sel = (jnp.arange(128, dtype=jnp.int32)[None, :]
               == lane[:, None]).astype(jnp.float32) * sign[:, None]

    ft_gather = _build_ft_gather(n_feat)
    sum_w, sum_b, prows = ft_gather(t_tab, idx_all)

    vmem = pl.BlockSpec(memory_space=pltpu.VMEM)
    smem = pl.BlockSpec(memory_space=pltpu.SMEM)
    out = pl.pallas_call(
        _tail_body,
        out_shape=jax.ShapeDtypeStruct((1, 1), jnp.float32),
        in_specs=[vmem, vmem, vmem, vmem, vmem, smem,
                  vmem, vmem, vmem, vmem, vmem, vmem],
        out_specs=vmem,
    )(
        sum_w.reshape(1, -1), sum_b.reshape(1, -1), prows, col_sel,
        ft_b.reshape(1, -1), stm.reshape(1, 1),
        l1_W, l1_b.reshape(1, -1), l2_W, l2_b.reshape(1, -1),
        out_W, out_b.reshape(1, 1),
    )
    return out.reshape(bsz)


# speedup vs baseline: 4.4895x; 1.0844x over previous
import functools

import jax
import jax.numpy as jnp
from jax import lax
from jax.experimental import pallas as pl
from jax.experimental.pallas import tpu as pltpu
from jax.experimental.pallas import tpu_sc as plsc

_FT_MAIN = 2048
_BAND = 64
_LANES = 16
_NBUF = 8


def _build_ft_gather(n_feat):
    mesh = plsc.VectorSubcoreMesh(core_axis_name="c", subcore_axis_name="s")
    psq_per_worker = 8
    psq_workers = n_feat // psq_per_worker

    @functools.partial(
        pl.kernel,
        mesh=mesh,
        compiler_params=pltpu.CompilerParams(needs_layout_passes=False),
        out_type=(
            jax.ShapeDtypeStruct((_FT_MAIN,), jnp.float32),
            jax.ShapeDtypeStruct((_FT_MAIN,), jnp.float32),
            jax.ShapeDtypeStruct((n_feat, 128), jnp.float32),
        ),
        scratch_types=[
            pltpu.VMEM((n_feat,), jnp.int32),
            pltpu.VMEM((n_feat,), jnp.int32),
            pltpu.VMEM((n_feat,), jnp.int32),
            pltpu.VMEM((n_feat,), jnp.float32),
            pltpu.VMEM((n_feat,), jnp.float32),
            pltpu.VMEM((_NBUF, _BAND, 128), jnp.float32),
            pltpu.VMEM((_BAND,), jnp.float32),
            pltpu.VMEM((_BAND,), jnp.float32),
            pltpu.VMEM((psq_per_worker, 128), jnp.float32),
            [pltpu.SemaphoreType.DMA] * _NBUF,
            pltpu.SemaphoreType.DMA,
        ],
    )
    def ft_gather(t_hbm, idx_hbm, out_w, out_b, out_psq,
                  idx_v, base_v, lane_v, wgt_w_v, wgt_b_v, buf_v,
                  acc_w_v, acc_b_v, prow_v, sems, psem):
        c = lax.axis_index("c")
        s = lax.axis_index("s")
        wid = c * 16 + s
        c0 = pl.multiple_of(wid * _BAND, _BAND)
        psq_last_row = t_hbm.shape[0] - 1

        pltpu.sync_copy(idx_hbm, idx_v)
        n_ch = n_feat // _LANES
        for k in range(n_ch):
            sl = pl.ds(k * _LANES, _LANES)
            ch = idx_v[sl]
            safe = jnp.maximum(ch, 0)
            maskf = jnp.where(ch >= 0, 1.0, 0.0).astype(jnp.float32)
            zerof = jnp.zeros((_LANES,), jnp.float32)
            base_v[sl] = safe & ~jnp.int32(127)
            lane_v[sl] = safe & jnp.int32(127)
            wgt_w_v[sl] = maskf if k < n_ch // 2 else zerof
            wgt_b_v[sl] = zerof if k < n_ch // 2 else maskf

        def _fetch_scalar(ref, j):
            return plsc.load_gather(ref, [jnp.full((_LANES,), j, jnp.int32)])[0]

        def _window_copy(j, slot):
            base_j = pl.multiple_of(_fetch_scalar(base_v, j), 128)
            pltpu.async_copy(
                t_hbm.at[pl.ds(c0, _BAND), pl.ds(base_j, 128)],
                buf_v.at[slot], sems[slot])

        zero = jnp.zeros((_LANES,), jnp.float32)
        for k in range(_BAND // _LANES):
            acc_w_v[pl.ds(k * _LANES, _LANES)] = zero
            acc_b_v[pl.ds(k * _LANES, _LANES)] = zero

        is_psq = s < psq_workers // 2
        pbase = (c * (psq_workers // 2) + s) * psq_per_worker

        @pl.when(is_psq)
        def _psq_start():
            for i in range(psq_per_worker):
                pb = pl.multiple_of(_fetch_scalar(base_v, pbase + i), 128)
                pltpu.async_copy(
                    t_hbm.at[pl.ds(psq_last_row, 1), pl.ds(pb, 128)],
                    prow_v.at[pl.ds(i, 1)], psem)

        for t in range(_NBUF):
            _window_copy(t, t)

        def _outer(i, _):
            j0 = i * _NBUF
            for t in range(_NBUF):
                j = j0 + t
                pltpu.make_async_copy(
                    t_hbm.at[pl.ds(c0, _BAND), pl.ds(0, 128)],
                    buf_v.at[t], sems[t]).wait()
                lane_j = _fetch_scalar(lane_v, j)
                cols = jnp.full((_LANES,), lane_j, jnp.int32)
                wgt_w_j = _fetch_scalar(wgt_w_v, j)
                wgt_b_j = _fetch_scalar(wgt_b_v, j)
                for k in range(_BAND // _LANES):
                    rows = lax.iota(jnp.int32, _LANES) + (k * _LANES)
                    vals = plsc.load_gather(buf_v.at[t], [rows, cols])
                    sl = pl.ds(k * _LANES, _LANES)
                    acc_w_v[sl] = acc_w_v[sl] + vals * wgt_w_j
                    acc_b_v[sl] = acc_b_v[sl] + vals * wgt_b_j

                @pl.when(j + _NBUF < n_feat)
                def _refill():
                    _window_copy(j + _NBUF, t)
            return _

        lax.fori_loop(0, n_feat // _NBUF, _outer, None)
        pltpu.sync_copy(acc_w_v, out_w.at[pl.ds(c0, _BAND)])
        pltpu.sync_copy(acc_b_v, out_b.at[pl.ds(c0, _BAND)])

        @pl.when(is_psq)
        def _psq_finish():
            for i in range(psq_per_worker):
                pltpu.make_async_copy(
                    t_hbm.at[pl.ds(psq_last_row, 1), pl.ds(0, 128)],
                    prow_v.at[pl.ds(i, 1)], psem).wait()
            pltpu.sync_copy(
                prow_v, out_psq.at[pl.ds(pbase, psq_per_worker)])

    return ft_gather


def _tail_body(sw_ref, sb_ref, prow_ref, sel_ref, ftb_ref, stm_ref,
               l1w_ref, l1b_ref, l2w_ref, l2b_ref, ow_ref, ob_ref, out_ref):
    d = _FT_MAIN
    stm = stm_ref[0, 0]
    w = sw_ref[...] + ftb_ref[:, :d]
    b = sb_ref[...] + ftb_ref[:, :d]
    acc = stm * jnp.concatenate([w, b], axis=1) \
        + (1.0 - stm) * jnp.concatenate([b, w], axis=1)
    l1 = jnp.clip(acc, 0.0, 1.0)
    h1 = lax.dot_general(l1, l1w_ref[...], (((1,), (1,)), ((), ())),
                         preferred_element_type=jnp.float32) + l1b_ref[...]
    h2 = lax.dot_general(h1, l2w_ref[...], (((1,), (1,)), ((), ())),
                         preferred_element_type=jnp.float32) + l2b_ref[...]
    h2 = jnp.clip(h2, 0.0, 1.0)
    out = jnp.sum(h2 * ow_ref[...], axis=1, keepdims=True) + ob_ref[...]
    psqt_diff = jnp.sum(prow_ref[...] * sel_ref[...])
    out_ref[...] = out + psqt_diff * (stm - 0.5)


def kernel(white_features, black_features, stm, piece_counts,
           ft_W, ft_b, l1_W, l1_b, l2_W, l2_b, out_W, out_b):
    del piece_counts
    in_dim, d_cols = ft_W.shape
    bsz, f = white_features.shape
    n_feat = 2 * f

    t_tab = ft_W.T

    widx = white_features.reshape(-1).astype(jnp.int32)
    bidx = black_features.reshape(-1).astype(jnp.int32)
    idx_all = jnp.concatenate([widx, bidx])
    safe_idx = jnp.maximum(idx_all, 0)
    mask = (idx_all >= 0).astype(jnp.float32)
    lane = safe_idx % 128

    sign = jnp.where(jnp.arange(n_feat) < f, 1.0, -1.0) * mask
    col_sel = (jnp.arange(128, dtype=jnp.int32)[None, :]
               == lane[:, None]).astype(jnp.float32) * sign[:, None]

    ft_gather = _build_ft_gather(n_feat)
    sum_w, sum_b, prows = ft_gather(t_tab, idx_all)

    vmem = pl.BlockSpec(memory_space=pltpu.VMEM)
    smem = pl.BlockSpec(memory_space=pltpu.SMEM)
    out = pl.pallas_call(
        _tail_body,
        out_shape=jax.ShapeDtypeStruct((1, 1), jnp.float32),
        in_specs=[vmem, vmem, vmem, vmem, vmem, smem,
                  vmem, vmem, vmem, vmem, vmem, vmem],
        out_specs=vmem,
    )(
        sum_w.reshape(1, -1), sum_b.reshape(1, -1), prows, col_sel,
        ft_b.reshape(1, -1), stm.reshape(1, 1),
        l1_W, l1_b.reshape(1, -1), l2_W, l2_b.reshape(1, -1),
        out_W, out_b.reshape(1, 1),
    )
    return out.reshape(bsz)


# speedup vs baseline: 4.5867x; 1.0217x over previous
import functools

import jax
import jax.numpy as jnp
from jax import lax
from jax.experimental import pallas as pl
from jax.experimental.pallas import tpu as pltpu
from jax.experimental.pallas import tpu_sc as plsc

_FT_MAIN = 2048
_BAND = 64
_LANES = 16
_NBUF = 8


def _build_ft_gather(n_feat):
    mesh = plsc.VectorSubcoreMesh(core_axis_name="c", subcore_axis_name="s")
    psq_per_worker = 8
    psq_workers = n_feat // psq_per_worker

    @functools.partial(
        pl.kernel,
        mesh=mesh,
        compiler_params=pltpu.CompilerParams(needs_layout_passes=False),
        out_type=(
            jax.ShapeDtypeStruct((_FT_MAIN,), jnp.float32),
            jax.ShapeDtypeStruct((_FT_MAIN,), jnp.float32),
            jax.ShapeDtypeStruct((n_feat, 128), jnp.float32),
        ),
        scratch_types=[
            pltpu.VMEM((n_feat,), jnp.int32),
            pltpu.VMEM((n_feat,), jnp.int32),
            pltpu.VMEM((n_feat,), jnp.int32),
            pltpu.VMEM((n_feat,), jnp.float32),
            pltpu.VMEM((n_feat,), jnp.float32),
            pltpu.VMEM((_NBUF, _BAND, 128), jnp.float32),
            pltpu.VMEM((_BAND,), jnp.float32),
            pltpu.VMEM((_BAND,), jnp.float32),
            pltpu.VMEM((psq_per_worker, 128), jnp.float32),
            [pltpu.SemaphoreType.DMA] * _NBUF,
            pltpu.SemaphoreType.DMA,
        ],
    )
    def ft_gather(t_hbm, idx_w_hbm, idx_b_hbm, out_w, out_b, out_psq,
                  idx_v, base_v, lane_v, wgt_w_v, wgt_b_v, buf_v,
                  acc_w_v, acc_b_v, prow_v, sems, psem):
        c = lax.axis_index("c")
        s = lax.axis_index("s")
        wid = c * 16 + s
        c0 = pl.multiple_of(wid * _BAND, _BAND)
        psq_last_row = t_hbm.shape[0] - 1

        pltpu.sync_copy(idx_w_hbm, idx_v.at[pl.ds(0, n_feat // 2)])
        pltpu.sync_copy(idx_b_hbm, idx_v.at[pl.ds(n_feat // 2, n_feat // 2)])
        n_ch = n_feat // _LANES
        for k in range(n_ch):
            sl = pl.ds(k * _LANES, _LANES)
            ch = idx_v[sl]
            safe = jnp.maximum(ch, 0)
            maskf = jnp.where(ch >= 0, 1.0, 0.0).astype(jnp.float32)
            zerof = jnp.zeros((_LANES,), jnp.float32)
            base_v[sl] = safe & ~jnp.int32(127)
            lane_v[sl] = safe & jnp.int32(127)
            wgt_w_v[sl] = maskf if k < n_ch // 2 else zerof
            wgt_b_v[sl] = zerof if k < n_ch // 2 else maskf

        def _fetch_scalar(ref, j):
            return plsc.load_gather(ref, [jnp.full((_LANES,), j, jnp.int32)])[0]

        def _window_copy(j, slot):
            base_j = pl.multiple_of(_fetch_scalar(base_v, j), 128)
            pltpu.async_copy(
                t_hbm.at[pl.ds(c0, _BAND), pl.ds(base_j, 128)],
                buf_v.at[slot], sems[slot])

        zero = jnp.zeros((_LANES,), jnp.float32)
        for k in range(_BAND // _LANES):
            acc_w_v[pl.ds(k * _LANES, _LANES)] = zero
            acc_b_v[pl.ds(k * _LANES, _LANES)] = zero

        is_psq = s < psq_workers // 2
        pbase = (c * (psq_workers // 2) + s) * psq_per_worker

        @pl.when(is_psq)
        def _psq_start():
            for i in range(psq_per_worker):
                pb = pl.multiple_of(_fetch_scalar(base_v, pbase + i), 128)
                pltpu.async_copy(
                    t_hbm.at[pl.ds(psq_last_row, 1), pl.ds(pb, 128)],
                    prow_v.at[pl.ds(i, 1)], psem)

        for t in range(_NBUF):
            _window_copy(t, t)

        def _outer(i, _):
            j0 = i * _NBUF
            for t in range(_NBUF):
                j = j0 + t
                pltpu.make_async_copy(
                    t_hbm.at[pl.ds(c0, _BAND), pl.ds(0, 128)],
                    buf_v.at[t], sems[t]).wait()
                lane_j = _fetch_scalar(lane_v, j)
                cols = jnp.full((_LANES,), lane_j, jnp.int32)
                wgt_w_j = _fetch_scalar(wgt_w_v, j)
                wgt_b_j = _fetch_scalar(wgt_b_v, j)
                for k in range(_BAND // _LANES):
                    rows = lax.iota(jnp.int32, _LANES) + (k * _LANES)
                    vals = plsc.load_gather(buf_v.at[t], [rows, cols])
                    sl = pl.ds(k * _LANES, _LANES)
                    acc_w_v[sl] = acc_w_v[sl] + vals * wgt_w_j
                    acc_b_v[sl] = acc_b_v[sl] + vals * wgt_b_j

                @pl.when(j + _NBUF < n_feat)
                def _refill():
                    _window_copy(j + _NBUF, t)
            return _

        lax.fori_loop(0, n_feat // _NBUF, _outer, None)
        pltpu.sync_copy(acc_w_v, out_w.at[pl.ds(c0, _BAND)])
        pltpu.sync_copy(acc_b_v, out_b.at[pl.ds(c0, _BAND)])

        @pl.when(is_psq)
        def _psq_finish():
            for i in range(psq_per_worker):
                pltpu.make_async_copy(
                    t_hbm.at[pl.ds(psq_last_row, 1), pl.ds(0, 128)],
                    prow_v.at[pl.ds(i, 1)], psem).wait()
            pltpu.sync_copy(
                prow_v, out_psq.at[pl.ds(pbase, psq_per_worker)])

    return ft_gather


def _tail_body(sw_ref, sb_ref, prow_ref, sel_ref, ftb_ref, stm_ref,
               l1w_ref, l1b_ref, l2w_ref, l2b_ref, ow_ref, ob_ref, out_ref):
    d = _FT_MAIN
    stm = stm_ref[0, 0]
    w = sw_ref[...] + ftb_ref[:, :d]
    b = sb_ref[...] + ftb_ref[:, :d]
    acc = stm * jnp.concatenate([w, b], axis=1) \
        + (1.0 - stm) * jnp.concatenate([b, w], axis=1)
    l1 = jnp.clip(acc, 0.0, 1.0)
    h1 = lax.dot_general(l1, l1w_ref[...], (((1,), (1,)), ((), ())),
                         preferred_element_type=jnp.float32) + l1b_ref[...]
    h2 = lax.dot_general(h1, l2w_ref[...], (((1,), (1,)), ((), ())),
                         preferred_element_type=jnp.float32) + l2b_ref[...]
    h2 = jnp.clip(h2, 0.0, 1.0)
    out = jnp.sum(h2 * ow_ref[...], axis=1, keepdims=True) + ob_ref[...]
    psqt_diff = jnp.sum(prow_ref[...] * sel_ref[...])
    out_ref[...] = out + psqt_diff * (stm - 0.5)


def kernel(white_features, black_features, stm, piece_counts,
           ft_W, ft_b, l1_W, l1_b, l2_W, l2_b, out_W, out_b):
    del piece_counts
    in_dim, d_cols = ft_W.shape
    bsz, f = white_features.shape
    n_feat = 2 * f

    t_tab = ft_W.T

    widx = white_features.reshape(-1).astype(jnp.int32)
    bidx = black_features.reshape(-1).astype(jnp.int32)
    idx_all = jnp.concatenate([widx, bidx])
    safe_idx = jnp.maximum(idx_all, 0)
    mask = (idx_all >= 0).astype(jnp.float32)
    lane = safe_idx % 128

    sign = jnp.where(jnp.arange(n_feat) < f, 1.0, -1.0) * mask
    col_sel = (jnp.arange(128, dtype=jnp.int32)[None, :]
               == lane[:, None]).astype(jnp.float32) * sign[:, None]

    ft_gather = _build_ft_gather(n_feat)
    sum_w, sum_b, prows = ft_gather(t_tab, widx, bidx)

    vmem = pl.BlockSpec(memory_space=pltpu.VMEM)
    smem = pl.BlockSpec(memory_space=pltpu.SMEM)
    out = pl.pallas_call(
        _tail_body,
        out_shape=jax.ShapeDtypeStruct((1, 1), jnp.float32),
        in_specs=[vmem, vmem, vmem, vmem, vmem, smem,
                  vmem, vmem, vmem, vmem, vmem, vmem],
        out_specs=vmem,
    )(
        sum_w.reshape(1, -1), sum_b.reshape(1, -1), prows, col_sel,
        ft_b.reshape(1, -1), stm.reshape(1, 1),
        l1_W, l1_b.reshape(1, -1), l2_W, l2_b.reshape(1, -1),
        out_W, out_b.reshape(1, 1),
    )
    return out.reshape(bsz)
